# baseline (speedup 1.0000x reference)
import functools

import jax
import jax.numpy as jnp
from jax import lax
from jax.experimental import pallas as pl
from jax.experimental.pallas import tpu as pltpu
from jax.experimental.pallas import tpu_sc as plsc

N = 10000
E = 320000
D_IN = 128
D_H = 64

NC = 2
NS = 16
NW = NC * NS
CHUNK = 80
NCH = 125
RPS = N // NS
ZR = 125

_mesh = plsc.VectorSubcoreMesh(
    core_axis_name="c", subcore_axis_name="s", num_cores=NC, num_subcores=NS
)


@functools.partial(
    pl.kernel,
    out_type=jax.ShapeDtypeStruct((NC, N, D_H), jnp.float32),
    mesh=_mesh,
    scratch_types=[
        pltpu.VMEM((NCH, CHUNK), jnp.int32),
        pltpu.VMEM((NCH, CHUNK), jnp.int32),
        pltpu.VMEM((CHUNK, D_H), jnp.float32),
        pltpu.VMEM((CHUNK, D_H), jnp.float32),
        pltpu.VMEM((ZR, D_H), jnp.float32),
        pltpu.VMEM_SHARED((N, D_H), jnp.float32),
        pltpu.SemaphoreType.DMA,
        pltpu.SemaphoreType.DMA,
    ],
    compiler_params=pltpu.CompilerParams(use_tc_tiling_on_sc=False),
)
def _sc_segment_sum(y_hbm, src_hbm, dst_hbm, out_hbm,
                    src_v, dst_v, buf_a, buf_b, zbuf, acc, sem_a, sem_b):
    cid = lax.axis_index("c")
    sid = lax.axis_index("s")
    wid = cid * NS + sid

    pltpu.async_copy(src_hbm.at[wid], src_v, sem_a)
    pltpu.async_copy(dst_hbm.at[wid], dst_v, sem_b)

    zv = jnp.zeros((16,), jnp.float32)

    @pl.loop(0, ZR)
    def _(r):
        @pl.loop(0, D_H, step=16)
        def _(c):
            zbuf.at[r, pl.ds(c, 16)][...] = zv

    @pl.loop(0, RPS, step=ZR)
    def _(r):
        pltpu.sync_copy(zbuf, acc.at[pl.ds(sid * RPS + r, ZR)])

    pltpu.make_async_copy(src_hbm.at[wid], src_v, sem_a).wait()
    pltpu.make_async_copy(dst_hbm.at[wid], dst_v, sem_b).wait()
    plsc.subcore_barrier()

    def gather(j, buf, sem):
        return pltpu.async_copy(y_hbm.at[src_v.at[j]], buf, sem)

    def scat_add(j, buf):
        pltpu.sync_copy(buf, acc.at[dst_v.at[j]], add=True)

    def wait(j, buf, sem):
        pltpu.make_async_copy(y_hbm.at[src_v.at[j]], buf, sem).wait()

    gather(0, buf_a, sem_a)

    @pl.loop(0, NCH - 1, step=2)
    def _(j):
        gather(j + 1, buf_b, sem_b)
        wait(j, buf_a, sem_a)
        scat_add(j, buf_a)
        gather(j + 2, buf_a, sem_a)
        wait(j + 1, buf_b, sem_b)
        scat_add(j + 1, buf_b)

    wait(NCH - 1, buf_a, sem_a)
    scat_add(NCH - 1, buf_a)

    plsc.subcore_barrier()
    rows = pl.ds(sid * RPS, RPS)
    pltpu.sync_copy(acc.at[rows], out_hbm.at[cid, rows])


def _mm1_body(x_ref, w_ref, o_ref):
    o_ref[...] = jnp.dot(x_ref[...], w_ref[...], preferred_element_type=jnp.float32)


_mm1 = pl.pallas_call(
    _mm1_body, out_shape=jax.ShapeDtypeStruct((N, D_H), jnp.float32)
)


def _mlp2_body(y_ref, par_ref, b1_ref, w2_ref, b2_ref, o_ref):
    h = y_ref[...] + par_ref[0] + par_ref[1] + b1_ref[...]
    h = jnp.maximum(h, 0.0)
    o_ref[...] = jnp.dot(h, w2_ref[...], preferred_element_type=jnp.float32) + b2_ref[...]


_mlp2 = pl.pallas_call(
    _mlp2_body, out_shape=jax.ShapeDtypeStruct((N, D_H), jnp.float32)
)


def kernel(x, edge_index, W1, b1, W2, b2):
    x = x.astype(jnp.float32)
    ei = edge_index.astype(jnp.int32)
    src = ei[0].reshape(NW, NCH, CHUNK)
    dst = ei[1].reshape(NW, NCH, CHUNK)

    y = _mm1(x, W1)
    partials = _sc_segment_sum(y, src, dst)
    return _mlp2(y, partials, b1.reshape(1, D_H), W2, b2.reshape(1, D_H))


# speedup vs baseline: 1.2192x; 1.2192x over previous
import functools

import jax
import jax.numpy as jnp
from jax import lax
from jax.experimental import pallas as pl
from jax.experimental.pallas import tpu as pltpu
from jax.experimental.pallas import tpu_sc as plsc

N = 10000
E = 320000
D_IN = 128
D_H = 64

NC = 2
NS = 16
NW = NC * NS
CHUNK = 80
NCH = 125
RPS = N // NS
ZR = 125
NSLOT = 10
HALF = NSLOT // 2

_mesh = plsc.VectorSubcoreMesh(
    core_axis_name="c", subcore_axis_name="s", num_cores=NC, num_subcores=NS
)


@functools.partial(
    pl.kernel,
    out_type=jax.ShapeDtypeStruct((NC, N, D_H), jnp.float32),
    mesh=_mesh,
    scratch_types=[
        pltpu.VMEM((NCH, CHUNK), jnp.int32),
        pltpu.VMEM((NCH, CHUNK), jnp.int32),
        [pltpu.VMEM((CHUNK, D_H), jnp.float32)] * NSLOT,
        pltpu.VMEM((ZR, D_H), jnp.float32),
        pltpu.VMEM_SHARED((N, D_H), jnp.float32),
        [pltpu.SemaphoreType.DMA] * NSLOT,
        [pltpu.SemaphoreType.DMA] * NSLOT,
    ],
    compiler_params=pltpu.CompilerParams(use_tc_tiling_on_sc=False),
)
def _sc_segment_sum(y_hbm, src_hbm, dst_hbm, out_hbm,
                    src_v, dst_v, bufs, zbuf, acc, gsems, ssems):
    cid = lax.axis_index("c")
    sid = lax.axis_index("s")
    wid = cid * NS + sid

    pltpu.async_copy(src_hbm.at[wid], src_v, gsems[0])
    pltpu.async_copy(dst_hbm.at[wid], dst_v, gsems[1])

    zv = jnp.zeros((16,), jnp.float32)

    @pl.loop(0, ZR)
    def _(r):
        @pl.loop(0, D_H, step=16)
        def _(c):
            zbuf.at[r, pl.ds(c, 16)][...] = zv

    @pl.loop(0, RPS, step=ZR)
    def _(r):
        pltpu.sync_copy(zbuf, acc.at[pl.ds(sid * RPS + r, ZR)])

    pltpu.make_async_copy(src_hbm.at[wid], src_v, gsems[0]).wait()
    pltpu.make_async_copy(dst_hbm.at[wid], dst_v, gsems[1]).wait()
    plsc.subcore_barrier()

    def gather(j, k):
        pltpu.async_copy(y_hbm.at[src_v.at[j]], bufs[k], gsems[k])

    def wait_gather(k):
        pltpu.make_async_copy(y_hbm.at[src_v.at[0]], bufs[k], gsems[k]).wait()

    def scat_add(j, k):
        pltpu.async_copy(bufs[k], acc.at[dst_v.at[j]], ssems[k], add=True)

    def wait_scat(k):
        pltpu.make_async_copy(bufs[k], acc.at[dst_v.at[0]], ssems[k]).wait()

    for t in range(HALF):
        gather(t, t)
    for t in range(HALF):
        wait_gather(t)
        scat_add(t, t)
        gather(t + HALF, t + HALF)

    @pl.loop(HALF, NCH - 2 * HALF, step=NSLOT)
    def _(j):
        for k in range(NSLOT):
            s = (HALF + k) % NSLOT
            p = k % NSLOT
            wait_gather(s)
            scat_add(j + k, s)
            wait_scat(p)
            gather(j + k + HALF, p)

    for t in range(NCH - 2 * HALF, NCH - HALF):
        s, p = t % NSLOT, (t + HALF) % NSLOT
        wait_gather(s)
        scat_add(t, s)
        wait_scat(p)
        gather(t + HALF, p)
    for t in range(NCH - HALF, NCH):
        s, p = t % NSLOT, (t + HALF) % NSLOT
        wait_gather(s)
        scat_add(t, s)
        wait_scat(p)
    for t in range(NCH - HALF, NCH):
        wait_scat(t % NSLOT)

    plsc.subcore_barrier()
    rows = pl.ds(sid * RPS, RPS)
    pltpu.sync_copy(acc.at[rows], out_hbm.at[cid, rows])


def _mm1_body(x_ref, w_ref, o_ref):
    o_ref[...] = jnp.dot(x_ref[...], w_ref[...], preferred_element_type=jnp.float32)


_mm1 = pl.pallas_call(
    _mm1_body, out_shape=jax.ShapeDtypeStruct((N, D_H), jnp.float32)
)


def _mlp2_body(y_ref, par_ref, b1_ref, w2_ref, b2_ref, o_ref):
    h = y_ref[...] + par_ref[0] + par_ref[1] + b1_ref[...]
    h = jnp.maximum(h, 0.0)
    o_ref[...] = jnp.dot(h, w2_ref[...], preferred_element_type=jnp.float32) + b2_ref[...]


_mlp2 = pl.pallas_call(
    _mlp2_body, out_shape=jax.ShapeDtypeStruct((N, D_H), jnp.float32)
)


def kernel(x, edge_index, W1, b1, W2, b2):
    x = x.astype(jnp.float32)
    ei = edge_index.astype(jnp.int32)
    src = ei[0].reshape(NW, NCH, CHUNK)
    dst = ei[1].reshape(NW, NCH, CHUNK)

    y = _mm1(x, W1)
    partials = _sc_segment_sum(y, src, dst)
    return _mlp2(y, partials, b1.reshape(1, D_H), W2, b2.reshape(1, D_H))


# speedup vs baseline: 1.3227x; 1.0849x over previous
import functools

import jax
import jax.numpy as jnp
from jax import lax
from jax.experimental import pallas as pl
from jax.experimental.pallas import tpu as pltpu
from jax.experimental.pallas import tpu_sc as plsc

N = 10000
E = 320000
D_IN = 128
D_H = 64

NC = 2
NS = 16
NW = NC * NS
CHUNK = 80
NCH = 125
RPS = N // NS
ZR = 125
NSLOT = 10
HALF = NSLOT // 2

_mesh = plsc.VectorSubcoreMesh(
    core_axis_name="c", subcore_axis_name="s", num_cores=NC, num_subcores=NS
)


@functools.partial(
    pl.kernel,
    out_type=jax.ShapeDtypeStruct((N, NC * D_H), jnp.float32),
    mesh=_mesh,
    scratch_types=[
        pltpu.VMEM((NCH, CHUNK), jnp.int32),
        pltpu.VMEM((NCH, CHUNK), jnp.int32),
        [pltpu.VMEM((CHUNK, D_H), jnp.float32)] * NSLOT,
        pltpu.VMEM((ZR, D_H), jnp.float32),
        pltpu.VMEM_SHARED((N, D_H), jnp.float32),
        [pltpu.SemaphoreType.DMA] * NSLOT,
        [pltpu.SemaphoreType.DMA] * NSLOT,
    ],
    compiler_params=pltpu.CompilerParams(use_tc_tiling_on_sc=False),
)
def _sc_segment_sum(y_hbm, src_hbm, dst_hbm, out_hbm,
                    src_v, dst_v, bufs, zbuf, acc, gsems, ssems):
    cid = lax.axis_index("c")
    sid = lax.axis_index("s")
    wid = cid * NS + sid

    pltpu.async_copy(src_hbm.at[wid], src_v, gsems[0])
    pltpu.async_copy(dst_hbm.at[wid], dst_v, gsems[1])

    zv = jnp.zeros((16,), jnp.float32)

    @pl.loop(0, ZR)
    def _(r):
        @pl.loop(0, D_H, step=16)
        def _(c):
            zbuf.at[r, pl.ds(c, 16)][...] = zv

    @pl.loop(0, RPS, step=ZR)
    def _(r):
        pltpu.sync_copy(zbuf, acc.at[pl.ds(sid * RPS + r, ZR)])

    pltpu.make_async_copy(src_hbm.at[wid], src_v, gsems[0]).wait()
    pltpu.make_async_copy(dst_hbm.at[wid], dst_v, gsems[1]).wait()
    plsc.subcore_barrier()

    def gather(j, k):
        pltpu.async_copy(y_hbm.at[src_v.at[j]], bufs[k], gsems[k])

    def wait_gather(k):
        pltpu.make_async_copy(y_hbm.at[src_v.at[0]], bufs[k], gsems[k]).wait()

    def scat_add(j, k):
        pltpu.async_copy(bufs[k], acc.at[dst_v.at[j]], ssems[k], add=True)

    def wait_scat(k):
        pltpu.make_async_copy(bufs[k], acc.at[dst_v.at[0]], ssems[k]).wait()

    for t in range(HALF):
        gather(t, t)
    for t in range(HALF):
        wait_gather(t)
        scat_add(t, t)
        gather(t + HALF, t + HALF)

    @pl.loop(HALF, NCH - 2 * HALF, step=NSLOT)
    def _(j):
        for k in range(NSLOT):
            s = (HALF + k) % NSLOT
            p = k % NSLOT
            wait_gather(s)
            scat_add(j + k, s)
            wait_scat(p)
            gather(j + k + HALF, p)

    for t in range(NCH - 2 * HALF, NCH - HALF):
        s, p = t % NSLOT, (t + HALF) % NSLOT
        wait_gather(s)
        scat_add(t, s)
        wait_scat(p)
        gather(t + HALF, p)
    for t in range(NCH - HALF, NCH):
        s, p = t % NSLOT, (t + HALF) % NSLOT
        wait_gather(s)
        scat_add(t, s)
        wait_scat(p)
    for t in range(NCH - HALF, NCH):
        wait_scat(t % NSLOT)

    plsc.subcore_barrier()
    rows = pl.ds(sid * RPS, RPS)
    pltpu.sync_copy(acc.at[rows], out_hbm.at[rows, pl.ds(cid * D_H, D_H)])


def _mm1_body(x_ref, w_ref, o_ref):
    o_ref[...] = jnp.dot(x_ref[...], w_ref[...], preferred_element_type=jnp.float32)


_mm1 = pl.pallas_call(
    _mm1_body, out_shape=jax.ShapeDtypeStruct((N, D_H), jnp.float32)
)


def _mlp2_body(y_ref, par_ref, b1_ref, w2_ref, b2_ref, o_ref):
    par = par_ref[...]
    h = y_ref[...] + par[:, :D_H] + par[:, D_H:] + b1_ref[...]
    h = jnp.maximum(h, 0.0)
    o_ref[...] = jnp.dot(h, w2_ref[...], preferred_element_type=jnp.float32) + b2_ref[...]


_mlp2 = pl.pallas_call(
    _mlp2_body, out_shape=jax.ShapeDtypeStruct((N, D_H), jnp.float32)
)


def kernel(x, edge_index, W1, b1, W2, b2):
    x = x.astype(jnp.float32)
    ei = edge_index.astype(jnp.int32)
    src = ei[0].reshape(NW, NCH, CHUNK)
    dst = ei[1].reshape(NW, NCH, CHUNK)

    y = _mm1(x, W1)
    partials = _sc_segment_sum(y, src, dst)
    return _mlp2(y, partials, b1.reshape(1, D_H), W2, b2.reshape(1, D_H))


# speedup vs baseline: 1.3697x; 1.0355x over previous
import functools

import jax
import jax.numpy as jnp
from jax import lax
from jax.experimental import pallas as pl
from jax.experimental.pallas import tpu as pltpu
from jax.experimental.pallas import tpu_sc as plsc

N = 10000
E = 320000
D_IN = 128
D_H = 64

NC = 2
NS = 16
NW = NC * NS
CHUNK = 80
NCH = 125
RPS = N // NS
ZR = 125
NSLOT = 10
HALF = NSLOT // 2

_mesh = plsc.VectorSubcoreMesh(
    core_axis_name="c", subcore_axis_name="s", num_cores=NC, num_subcores=NS
)


@functools.partial(
    pl.kernel,
    out_type=jax.ShapeDtypeStruct((N, NC * D_H), jnp.float32),
    mesh=_mesh,
    scratch_types=[
        pltpu.VMEM((NCH, CHUNK), jnp.int32),
        pltpu.VMEM((NCH, CHUNK), jnp.int32),
        [pltpu.VMEM((CHUNK, D_H), jnp.float32)] * NSLOT,
        pltpu.VMEM((ZR, D_H), jnp.float32),
        pltpu.VMEM_SHARED((N, D_H), jnp.float32),
        [pltpu.SemaphoreType.DMA] * NSLOT,
        [pltpu.SemaphoreType.DMA] * NSLOT,
    ],
    compiler_params=pltpu.CompilerParams(use_tc_tiling_on_sc=False),
)
def _sc_segment_sum(y_hbm, src_hbm, dst_hbm, out_hbm,
                    src_v, dst_v, bufs, zbuf, acc, gsems, ssems):
    cid = lax.axis_index("c")
    sid = lax.axis_index("s")
    wid = cid * NS + sid

    pltpu.async_copy(src_hbm.at[wid], src_v, gsems[0])
    pltpu.async_copy(dst_hbm.at[wid], dst_v, gsems[1])

    zv = jnp.zeros((16,), jnp.float32)

    @pl.loop(0, ZR)
    def _(r):
        @pl.loop(0, D_H, step=16)
        def _(c):
            zbuf.at[r, pl.ds(c, 16)][...] = zv

    @pl.loop(0, RPS, step=ZR)
    def _(r):
        pltpu.sync_copy(zbuf, acc.at[pl.ds(sid * RPS + r, ZR)])

    pltpu.make_async_copy(src_hbm.at[wid], src_v, gsems[0]).wait()
    pltpu.make_async_copy(dst_hbm.at[wid], dst_v, gsems[1]).wait()
    plsc.subcore_barrier()

    def gather(j, k):
        pltpu.async_copy(y_hbm.at[src_v.at[j]], bufs[k], gsems[k])

    def wait_gather(k):
        pltpu.make_async_copy(y_hbm.at[src_v.at[0]], bufs[k], gsems[k]).wait()

    def scat_add(j, k):
        pltpu.async_copy(bufs[k], acc.at[dst_v.at[j]], ssems[k], add=True)

    def wait_scat(k):
        pltpu.make_async_copy(bufs[k], acc.at[dst_v.at[0]], ssems[k]).wait()

    for t in range(HALF):
        gather(t, t)
    for t in range(HALF):
        wait_gather(t)
        scat_add(t, t)
        gather(t + HALF, t + HALF)

    @pl.loop(HALF, NCH - 2 * HALF, step=NSLOT)
    def _(j):
        for k in range(NSLOT):
            s = (HALF + k) % NSLOT
            p = k % NSLOT
            wait_gather(s)
            scat_add(j + k, s)
            wait_scat(p)
            gather(j + k + HALF, p)

    for t in range(NCH - 2 * HALF, NCH - HALF):
        s, p = t % NSLOT, (t + HALF) % NSLOT
        wait_gather(s)
        scat_add(t, s)
        wait_scat(p)
        gather(t + HALF, p)
    for t in range(NCH - HALF, NCH):
        s, p = t % NSLOT, (t + HALF) % NSLOT
        wait_gather(s)
        scat_add(t, s)
        wait_scat(p)
    for t in range(NCH - HALF, NCH):
        wait_scat(t % NSLOT)

    plsc.subcore_barrier()
    rows = pl.ds(sid * RPS, RPS)
    pltpu.sync_copy(acc.at[rows], out_hbm.at[rows, pl.ds(cid * D_H, D_H)])


def _mm1_body(x_ref, w_ref, o_ref):
    h = jnp.dot(x_ref[...], w_ref[...], preferred_element_type=jnp.float32)
    o_ref[...] = jnp.pad(h, ((0, 0), (0, D_H)))


_mm1 = pl.pallas_call(
    _mm1_body, out_shape=jax.ShapeDtypeStruct((N, 2 * D_H), jnp.float32)
)


def _mlp2_body(y_ref, par_ref, b1_ref, w2_ref, b2_ref, o_ref):
    par = par_ref[...]
    h = y_ref[:, :D_H] + par[:, :D_H] + par[:, D_H:] + b1_ref[...]
    h = jnp.maximum(h, 0.0)
    o_ref[...] = jnp.dot(h, w2_ref[...], preferred_element_type=jnp.float32) + b2_ref[...]


_mlp2 = pl.pallas_call(
    _mlp2_body, out_shape=jax.ShapeDtypeStruct((N, D_H), jnp.float32)
)


def kernel(x, edge_index, W1, b1, W2, b2):
    x = x.astype(jnp.float32)
    ei = edge_index.astype(jnp.int32)
    src = (ei[0] * 2).reshape(NW, NCH, CHUNK)
    dst = ei[1].reshape(NW, NCH, CHUNK)

    y2 = _mm1(x, W1)
    partials = _sc_segment_sum(y2.reshape(2 * N, D_H), src, dst)
    return _mlp2(y2, partials, b1.reshape(1, D_H), W2, b2.reshape(1, D_H))


# speedup vs baseline: 1.3729x; 1.0024x over previous
import functools

import jax
import jax.numpy as jnp
from jax import lax
from jax.experimental import pallas as pl
from jax.experimental.pallas import tpu as pltpu
from jax.experimental.pallas import tpu_sc as plsc

N = 10000
E = 320000
D_IN = 128
D_H = 64

NC = 2
NS = 16
NW = NC * NS
CHUNK = 80
NCH = 125
RPS = N // NS
ZR = 125
NSLOT = 10
HALF = NSLOT // 2

_mesh = plsc.VectorSubcoreMesh(
    core_axis_name="c", subcore_axis_name="s", num_cores=NC, num_subcores=NS
)


@functools.partial(
    pl.kernel,
    out_type=jax.ShapeDtypeStruct((N, NC * D_H), jnp.float32),
    mesh=_mesh,
    scratch_types=[
        pltpu.VMEM((NCH, CHUNK), jnp.int32),
        pltpu.VMEM((NCH, CHUNK), jnp.int32),
        [pltpu.VMEM((CHUNK, D_H), jnp.float32)] * NSLOT,
        pltpu.VMEM((ZR, D_H), jnp.float32),
        pltpu.VMEM_SHARED((N, D_H), jnp.float32),
        [pltpu.SemaphoreType.DMA] * NSLOT,
        [pltpu.SemaphoreType.DMA] * NSLOT,
    ],
    compiler_params=pltpu.CompilerParams(use_tc_tiling_on_sc=False),
)
def _sc_segment_sum(y_hbm, src_hbm, dst_hbm, out_hbm,
                    src_v, dst_v, bufs, zbuf, acc, gsems, ssems):
    cid = lax.axis_index("c")
    sid = lax.axis_index("s")
    wid = cid * NS + sid

    pltpu.async_copy(src_hbm.at[wid], src_v, gsems[0])
    pltpu.async_copy(dst_hbm.at[wid], dst_v, gsems[1])

    zv = jnp.zeros((16,), jnp.float32)

    @pl.loop(0, ZR)
    def _(r):
        @pl.loop(0, D_H, step=16)
        def _(c):
            zbuf.at[r, pl.ds(c, 16)][...] = zv

    @pl.loop(0, RPS, step=ZR)
    def _(r):
        pltpu.sync_copy(zbuf, acc.at[pl.ds(sid * RPS + r, ZR)])

    pltpu.make_async_copy(src_hbm.at[wid], src_v, gsems[0]).wait()
    pltpu.make_async_copy(dst_hbm.at[wid], dst_v, gsems[1]).wait()
    plsc.subcore_barrier()

    def gather(j, k):
        pltpu.async_copy(y_hbm.at[src_v.at[j]], bufs[k], gsems[k])

    def wait_gather(k):
        pltpu.make_async_copy(y_hbm.at[src_v.at[0]], bufs[k], gsems[k]).wait()

    def scat_add(j, k):
        pltpu.async_copy(bufs[k], acc.at[dst_v.at[j]], ssems[k], add=True)

    def wait_scat(k):
        pltpu.make_async_copy(bufs[k], acc.at[dst_v.at[0]], ssems[k]).wait()

    for t in range(HALF):
        gather(t, t)
    for t in range(HALF):
        wait_gather(t)
        scat_add(t, t)
        gather(t + HALF, t + HALF)

    @pl.loop(HALF, NCH - 2 * HALF, step=NSLOT)
    def _(j):
        for k in range(NSLOT):
            s = (HALF + k) % NSLOT
            p = k % NSLOT
            wait_gather(s)
            scat_add(j + k, s)
            wait_scat(p)
            gather(j + k + HALF, p)

    for t in range(NCH - 2 * HALF, NCH - HALF):
        s, p = t % NSLOT, (t + HALF) % NSLOT
        wait_gather(s)
        scat_add(t, s)
        wait_scat(p)
        gather(t + HALF, p)
    for t in range(NCH - HALF, NCH):
        s, p = t % NSLOT, (t + HALF) % NSLOT
        wait_gather(s)
        scat_add(t, s)
        wait_scat(p)
    for t in range(NCH - HALF, NCH):
        wait_scat(t % NSLOT)

    plsc.subcore_barrier()
    rows = pl.ds(sid * RPS, RPS)
    pltpu.sync_copy(acc.at[rows], out_hbm.at[rows, pl.ds(cid * D_H, D_H)])


def _mm1_body(x_ref, w_ref, o_ref):
    h = jnp.dot(x_ref[...], w_ref[...], preferred_element_type=jnp.float32)
    o_ref[...] = jnp.pad(h, ((0, 0), (0, D_H)))


_mm1 = pl.pallas_call(
    _mm1_body, out_shape=jax.ShapeDtypeStruct((N, 2 * D_H), jnp.float32)
)


def _mlp2_body(y_ref, par_ref, b1_ref, w2_ref, b2_ref, o_ref):
    par = par_ref[...]
    h = y_ref[:, :D_H] + par[:, :D_H] + par[:, D_H:] + b1_ref[...]
    h = jnp.maximum(h, 0.0)
    o_ref[...] = jnp.dot(h, w2_ref[...], preferred_element_type=jnp.float32) + b2_ref[...]


_mlp2 = pl.pallas_call(
    _mlp2_body, out_shape=jax.ShapeDtypeStruct((N, D_H), jnp.float32)
)


def kernel(x, edge_index, W1, b1, W2, b2):
    x = x.astype(jnp.float32)
    ei = edge_index.astype(jnp.int32)
    src_t, dst_t = lax.optimization_barrier(
        ((ei[0] * 2).reshape(E // 128, 128), ei[1].reshape(E // 128, 128)))
    src = src_t.reshape(NW, NCH, CHUNK)
    dst = dst_t.reshape(NW, NCH, CHUNK)

    y2 = _mm1(x, W1)
    partials = _sc_segment_sum(y2.reshape(2 * N, D_H), src, dst)
    return _mlp2(y2, partials, b1.reshape(1, D_H), W2, b2.reshape(1, D_H))
